# Initial kernel scaffold; baseline (speedup 1.0000x reference)
import jax
import jax.numpy as jnp
from jax.experimental import pallas as pl


def kernel(boxes, scores):
    raise NotImplementedError("write your pallas kernel here")


# speedup vs baseline: 396.8739x; 396.8739x over previous
import jax
import jax.numpy as jnp
from jax.experimental import pallas as pl
from jax.experimental.pallas import tpu as pltpu

IOU_THRES = 0.1
_LANES = 128


def _nms_body(cx_ref, cy_ref, w_ref, h_ref, s_ref, nvalid_ref, out_ref,
              x1_ref, y1_ref, x2_ref, y2_ref, area_ref, ms_ref):
    rows, lanes = cx_ref.shape
    cx = cx_ref[...]
    cy = cy_ref[...]
    w = w_ref[...]
    h = h_ref[...]
    x1 = cx - w / 2.0
    y1 = cy - h / 2.0
    x2 = cx + w / 2.0
    y2 = cy + h / 2.0
    x1_ref[...] = x1
    y1_ref[...] = y1
    x2_ref[...] = x2
    y2_ref[...] = y2
    area_ref[...] = (x2 - x1) * (y2 - y1)

    r = jax.lax.broadcasted_iota(jnp.int32, (rows, lanes), 0)
    c = jax.lax.broadcasted_iota(jnp.int32, (rows, lanes), 1)
    lin = r * lanes + c
    valid = lin < nvalid_ref[0]
    neg_inf = jnp.float32(-jnp.inf)
    ms_ref[...] = jnp.where(valid, s_ref[...], neg_inf)
    out_ref[...] = jnp.zeros((rows, lanes), jnp.float32)

    big = jnp.int32(2**31 - 1)

    def body(m):
        ms = ms_ref[...]
        win = jnp.min(jnp.where(ms == m, lin, big))
        onehot = lin == win
        zero = jnp.float32(0.0)
        bx1 = jnp.sum(jnp.where(onehot, x1_ref[...], zero))
        by1 = jnp.sum(jnp.where(onehot, y1_ref[...], zero))
        bx2 = jnp.sum(jnp.where(onehot, x2_ref[...], zero))
        by2 = jnp.sum(jnp.where(onehot, y2_ref[...], zero))
        barea = jnp.sum(jnp.where(onehot, area_ref[...], zero))
        ix1 = jnp.maximum(bx1, x1_ref[...])
        iy1 = jnp.maximum(by1, y1_ref[...])
        ix2 = jnp.minimum(bx2, x2_ref[...])
        iy2 = jnp.minimum(by2, y2_ref[...])
        inter = jnp.maximum(ix2 - ix1, zero) * jnp.maximum(iy2 - iy1, zero)
        iou = inter / (barea + area_ref[...] - inter + 1e-9)
        sup = (iou > IOU_THRES) | onehot
        new_ms = jnp.where(sup, neg_inf, ms)
        ms_ref[...] = new_ms
        out_ref[...] = jnp.where(onehot, s_ref[...], out_ref[...])
        return jnp.max(new_ms)

    m0 = jnp.max(ms_ref[...])
    jax.lax.while_loop(lambda m: m > neg_inf, body, m0)


def kernel(boxes, scores):
    n = scores.shape[0]
    rows = pl.cdiv(n, _LANES * 8) * 8
    npad = rows * _LANES
    pad = npad - n
    b = jnp.pad(boxes, ((0, pad), (0, 0)))
    s = jnp.pad(scores, (0, pad)).reshape(rows, _LANES)
    cx = b[:, 0].reshape(rows, _LANES)
    cy = b[:, 1].reshape(rows, _LANES)
    w = b[:, 2].reshape(rows, _LANES)
    h = b[:, 3].reshape(rows, _LANES)
    nvalid = jnp.array([n], dtype=jnp.int32)

    shape = jax.ShapeDtypeStruct((rows, _LANES), jnp.float32)
    out = pl.pallas_call(
        _nms_body,
        out_shape=shape,
        in_specs=[
            pl.BlockSpec(memory_space=pltpu.VMEM),
            pl.BlockSpec(memory_space=pltpu.VMEM),
            pl.BlockSpec(memory_space=pltpu.VMEM),
            pl.BlockSpec(memory_space=pltpu.VMEM),
            pl.BlockSpec(memory_space=pltpu.VMEM),
            pl.BlockSpec(memory_space=pltpu.SMEM),
        ],
        out_specs=pl.BlockSpec(memory_space=pltpu.VMEM),
        scratch_shapes=[pltpu.VMEM((rows, _LANES), jnp.float32)] * 6,
    )(cx, cy, w, h, s, nvalid)
    return out.reshape(-1)[:n]
